# Initial kernel scaffold; baseline (speedup 1.0000x reference)
import jax
import jax.numpy as jnp
from jax.experimental import pallas as pl


def kernel(x, table):
    raise NotImplementedError("write your pallas kernel here")


# speedup vs baseline: 1.6860x; 1.6860x over previous
import functools

import jax
import jax.numpy as jnp
from jax import lax
from jax.experimental import pallas as pl
from jax.experimental.pallas import tpu as pltpu
from jax.experimental.pallas import tpu_sc as plsc

CHUNK = 128


@functools.partial(jax.jit, static_argnames=())
def _emb_lookup(idx3, table):
    NW = idx3.shape[0]
    n_chunks = idx3.shape[1]
    B = NW * n_chunks * CHUNK
    D = table.shape[1]
    per_w = n_chunks * CHUNK

    mesh = plsc.VectorSubcoreMesh(core_axis_name="c", subcore_axis_name="s")

    @functools.partial(
        pl.kernel,
        out_type=jax.ShapeDtypeStruct((B, D), table.dtype),
        mesh=mesh,
        compiler_params=pltpu.CompilerParams(use_tc_tiling_on_sc=False),
        scratch_types=[
            pltpu.VMEM((n_chunks, CHUNK), jnp.int32),
            pltpu.VMEM((CHUNK, D), jnp.float32),
            pltpu.SemaphoreType.DMA,
        ],
    )
    def emb(idx_hbm, table_hbm, out_hbm, idx_v, rows_v, sem):
        wid = lax.axis_index("s") * 2 + lax.axis_index("c")
        base = wid * per_w
        pltpu.sync_copy(idx_hbm.at[wid], idx_v)

        def body(j, carry):
            pltpu.async_copy(table_hbm.at[idx_v.at[j]], rows_v, sem).wait()
            pltpu.sync_copy(rows_v, out_hbm.at[pl.ds(base + j * CHUNK, CHUNK)])
            return carry

        lax.fori_loop(0, n_chunks, body, 0, unroll=False)

    return emb(idx3, table)


def kernel(x, table):
    B0, S = x.shape
    NW = 32
    idx = x.reshape(-1).astype(jnp.int32)
    B = idx.shape[0]
    assert B % (NW * CHUNK) == 0
    idx3 = idx.reshape(NW, B // (NW * CHUNK), CHUNK)
    out = _emb_lookup(idx3, table)
    return out.reshape(B0, S, table.shape[1])


# speedup vs baseline: 1.8781x; 1.1139x over previous
import functools

import jax
import jax.numpy as jnp
from jax import lax
from jax.experimental import pallas as pl
from jax.experimental.pallas import tpu as pltpu
from jax.experimental.pallas import tpu_sc as plsc

CHUNK = 128
NBUF = 8


def _emb_lookup(idx3, table):
    NW = idx3.shape[0]
    n_chunks = idx3.shape[1]
    B = NW * n_chunks * CHUNK
    D = table.shape[1]
    per_w = n_chunks * CHUNK
    assert n_chunks % NBUF == 0 and n_chunks // NBUF >= 2

    mesh = plsc.VectorSubcoreMesh(core_axis_name="c", subcore_axis_name="s")

    @functools.partial(
        pl.kernel,
        out_type=jax.ShapeDtypeStruct((B, D), table.dtype),
        mesh=mesh,
        compiler_params=pltpu.CompilerParams(use_tc_tiling_on_sc=False),
        scratch_types=[
            pltpu.VMEM((n_chunks, CHUNK), jnp.int32),
            [pltpu.VMEM((CHUNK, D), jnp.float32) for _ in range(NBUF)],
            [pltpu.SemaphoreType.DMA for _ in range(NBUF)],
        ],
    )
    def emb(idx_hbm, table_hbm, out_hbm, idx_v, bufs, sems):
        wid = lax.axis_index("s") * 2 + lax.axis_index("c")
        base = wid * per_w
        pltpu.sync_copy(idx_hbm.at[wid], idx_v)

        for b in range(NBUF):
            pltpu.async_copy(table_hbm.at[idx_v.at[b]], bufs[b], sems[b])

        def wait_gather(b):
            pltpu.make_async_copy(
                table_hbm.at[pl.ds(0, CHUNK)], bufs[b], sems[b]
            ).wait()

        def outer(k, carry):
            j0 = k * NBUF
            for b in range(NBUF):
                j = j0 + b
                wait_gather(b)
                pltpu.sync_copy(bufs[b], out_hbm.at[pl.ds(base + j * CHUNK, CHUNK)])
                pltpu.async_copy(table_hbm.at[idx_v.at[j + NBUF]], bufs[b], sems[b])
            return carry

        lax.fori_loop(0, n_chunks // NBUF - 1, outer, 0, unroll=False)

        for b in range(NBUF):
            j = n_chunks - NBUF + b
            wait_gather(b)
            pltpu.sync_copy(bufs[b], out_hbm.at[pl.ds(base + j * CHUNK, CHUNK)])

    return emb(idx3, table)


def kernel(x, table):
    B0, S = x.shape
    NW = 32
    idx = x.reshape(-1).astype(jnp.int32)
    B = idx.shape[0]
    assert B % (NW * CHUNK) == 0
    idx3 = idx.reshape(NW, B // (NW * CHUNK), CHUNK)
    out = _emb_lookup(idx3, table)
    return out.reshape(B0, S, table.shape[1])
